# Initial kernel scaffold; baseline (speedup 1.0000x reference)
import jax
import jax.numpy as jnp
from jax.experimental import pallas as pl


def kernel(mu, t, gamma, W1, b1, W2, b2):
    raise NotImplementedError("write your pallas kernel here")


# speedup vs baseline: 3.7448x; 3.7448x over previous
import jax
import jax.numpy as jnp
from jax.experimental import pallas as pl
from jax.experimental.pallas import tpu as pltpu

_B, _D, _H, _K = 2048, 4096, 1024, 16
_SLOPE = 0.01
_TOP = 2.0 * (_K - 1.0) / _K - 1.0
_SQRT2 = 1.4142135623730951

_BM = 256
_DN = 512

_PAR = pltpu.GridDimensionSemantics.PARALLEL


def _mlp1_kernel(mu_ref, t_ref, w1_ref, w1r_ref, b1_ref, h_ref):
    acc = jnp.dot(mu_ref[...], w1_ref[...], preferred_element_type=jnp.float32)
    acc = acc + t_ref[...] * w1r_ref[...] + b1_ref[...]
    h_ref[...] = jnp.where(acc >= 0.0, acc, _SLOPE * acc)


def _out_kernel(h_ref, w2a_ref, w2b_ref, b2a_ref, b2b_ref, mu_ref, g_ref, o_ref):
    h = h_ref[...]
    mu_eps = jnp.dot(h, w2a_ref[...], preferred_element_type=jnp.float32) + b2a_ref[...]
    lse = jnp.dot(h, w2b_ref[...], preferred_element_type=jnp.float32) + b2b_ref[...]
    g = g_ref[...]
    s = jnp.sqrt((1.0 - g) / g)
    mu_x = mu_ref[...] / g - s * mu_eps
    inv = 1.0 / (s * jnp.exp(lse) * _SQRT2)
    o_ref[...] = 0.5 * (1.0 + jax.lax.erf((_TOP - mu_x) * inv))


def kernel(mu, t, gamma, W1, b1, W2, b2):
    w1m = W1[:_D]
    w1r = W1[_D:].reshape(1, _H)
    b1r = b1.reshape(1, _H)
    w2a = W2[:, :_D]
    w2b = W2[:, _D:]
    b2a = b2[:_D].reshape(1, _D)
    b2b = b2[_D:].reshape(1, _D)

    h = pl.pallas_call(
        _mlp1_kernel,
        grid=(_B // _BM,),
        in_specs=[
            pl.BlockSpec((_BM, _D), lambda i: (i, 0)),
            pl.BlockSpec((_BM, 1), lambda i: (i, 0)),
            pl.BlockSpec((_D, _H), lambda i: (0, 0)),
            pl.BlockSpec((1, _H), lambda i: (0, 0)),
            pl.BlockSpec((1, _H), lambda i: (0, 0)),
        ],
        out_specs=pl.BlockSpec((_BM, _H), lambda i: (i, 0)),
        out_shape=jax.ShapeDtypeStruct((_B, _H), jnp.float32),
        compiler_params=pltpu.CompilerParams(
            dimension_semantics=(_PAR,),
            vmem_limit_bytes=56 * 1024 * 1024,
        ),
    )(mu, t, w1m, w1r, b1r)

    out = pl.pallas_call(
        _out_kernel,
        grid=(_D // _DN,),
        in_specs=[
            pl.BlockSpec((_B, _H), lambda j: (0, 0)),
            pl.BlockSpec((_H, _DN), lambda j: (0, j)),
            pl.BlockSpec((_H, _DN), lambda j: (0, j)),
            pl.BlockSpec((1, _DN), lambda j: (0, j)),
            pl.BlockSpec((1, _DN), lambda j: (0, j)),
            pl.BlockSpec((_B, _DN), lambda j: (0, j)),
            pl.BlockSpec((_B, 1), lambda j: (0, 0)),
        ],
        out_specs=pl.BlockSpec((_B, _DN), lambda j: (0, j)),
        out_shape=jax.ShapeDtypeStruct((_B, _D), jnp.float32),
        compiler_params=pltpu.CompilerParams(
            dimension_semantics=(_PAR,),
            vmem_limit_bytes=56 * 1024 * 1024,
        ),
    )(h, w2a, w2b, b2a, b2b, mu, gamma)

    return out


# speedup vs baseline: 3.7739x; 1.0078x over previous
import jax
import jax.numpy as jnp
from jax.experimental import pallas as pl
from jax.experimental.pallas import tpu as pltpu

_B, _D, _H, _K = 2048, 4096, 1024, 16
_SLOPE = 0.01
_TOP = 2.0 * (_K - 1.0) / _K - 1.0
_SQRT2 = 1.4142135623730951

_BM = 256
_DN = 512

_PAR = pltpu.GridDimensionSemantics.PARALLEL


def _mlp1_kernel(mu_ref, t_ref, w1_ref, w1r_ref, b1_ref, h_ref):
    acc = jnp.dot(mu_ref[...].astype(jnp.bfloat16),
                  w1_ref[...].astype(jnp.bfloat16),
                  preferred_element_type=jnp.float32)
    acc = acc + t_ref[...] * w1r_ref[...] + b1_ref[...]
    h_ref[...] = jnp.where(acc >= 0.0, acc, _SLOPE * acc).astype(jnp.bfloat16)


def _out_kernel(h_ref, w2a_ref, w2b_ref, b2a_ref, b2b_ref, mu_ref, g_ref, o_ref):
    h = h_ref[...]
    mu_eps = jnp.dot(h, w2a_ref[...].astype(jnp.bfloat16),
                     preferred_element_type=jnp.float32) + b2a_ref[...]
    lse = jnp.dot(h, w2b_ref[...].astype(jnp.bfloat16),
                  preferred_element_type=jnp.float32) + b2b_ref[...]
    g = g_ref[...]
    s = jnp.sqrt((1.0 - g) / g)
    mu_x = mu_ref[...] / g - s * mu_eps
    inv = 1.0 / (s * jnp.exp(lse) * _SQRT2)
    o_ref[...] = 0.5 * (1.0 + jax.lax.erf((_TOP - mu_x) * inv))


def kernel(mu, t, gamma, W1, b1, W2, b2):
    w1m = W1[:_D]
    w1r = W1[_D:].reshape(1, _H)
    b1r = b1.reshape(1, _H)
    w2a = W2[:, :_D]
    w2b = W2[:, _D:]
    b2a = b2[:_D].reshape(1, _D)
    b2b = b2[_D:].reshape(1, _D)

    h = pl.pallas_call(
        _mlp1_kernel,
        grid=(_B // _BM,),
        in_specs=[
            pl.BlockSpec((_BM, _D), lambda i: (i, 0)),
            pl.BlockSpec((_BM, 1), lambda i: (i, 0)),
            pl.BlockSpec((_D, _H), lambda i: (0, 0)),
            pl.BlockSpec((1, _H), lambda i: (0, 0)),
            pl.BlockSpec((1, _H), lambda i: (0, 0)),
        ],
        out_specs=pl.BlockSpec((_BM, _H), lambda i: (i, 0)),
        out_shape=jax.ShapeDtypeStruct((_B, _H), jnp.bfloat16),
        compiler_params=pltpu.CompilerParams(
            dimension_semantics=(_PAR,),
            vmem_limit_bytes=56 * 1024 * 1024,
        ),
    )(mu, t, w1m, w1r, b1r)

    out = pl.pallas_call(
        _out_kernel,
        grid=(_D // _DN,),
        in_specs=[
            pl.BlockSpec((_B, _H), lambda j: (0, 0)),
            pl.BlockSpec((_H, _DN), lambda j: (0, j)),
            pl.BlockSpec((_H, _DN), lambda j: (0, j)),
            pl.BlockSpec((1, _DN), lambda j: (0, j)),
            pl.BlockSpec((1, _DN), lambda j: (0, j)),
            pl.BlockSpec((_B, _DN), lambda j: (0, j)),
            pl.BlockSpec((_B, 1), lambda j: (0, 0)),
        ],
        out_specs=pl.BlockSpec((_B, _DN), lambda j: (0, j)),
        out_shape=jax.ShapeDtypeStruct((_B, _D), jnp.float32),
        compiler_params=pltpu.CompilerParams(
            dimension_semantics=(_PAR,),
            vmem_limit_bytes=56 * 1024 * 1024,
        ),
    )(h, w2a, w2b, b2a, b2b, mu, gamma)

    return out


# speedup vs baseline: 4.8131x; 1.2754x over previous
import jax
import jax.numpy as jnp
from jax.experimental import pallas as pl
from jax.experimental.pallas import tpu as pltpu

_B, _D, _H, _K = 2048, 4096, 1024, 16
_SLOPE = 0.01
_TOP = 2.0 * (_K - 1.0) / _K - 1.0
_SQRT2 = 1.4142135623730951

_BM = 256
_DN = 512

_PAR = pltpu.GridDimensionSemantics.PARALLEL


def _mlp1_kernel(mu_ref, t_ref, w1_ref, w1r_ref, b1_ref, h_ref):
    acc = jnp.dot(mu_ref[...].astype(jnp.bfloat16), w1_ref[...],
                  preferred_element_type=jnp.float32)
    acc = acc + t_ref[...] * w1r_ref[...] + b1_ref[...]
    h_ref[...] = jnp.where(acc >= 0.0, acc, _SLOPE * acc).astype(jnp.bfloat16)


def _out_kernel(h_ref, w2a_ref, w2b_ref, b2a_ref, b2b_ref, mu_ref, g_ref, o_ref):
    h = h_ref[...]
    mu_eps = jnp.dot(h, w2a_ref[...].astype(jnp.bfloat16),
                     preferred_element_type=jnp.float32) + b2a_ref[...]
    lse = jnp.dot(h, w2b_ref[...].astype(jnp.bfloat16),
                  preferred_element_type=jnp.float32) + b2b_ref[...]
    g = g_ref[...]
    s = jnp.sqrt((1.0 - g) / g)
    mu_x = mu_ref[...] / g - s * mu_eps
    inv = 1.0 / (s * jnp.exp(lse) * _SQRT2)
    o_ref[...] = 0.5 * (1.0 + jax.lax.erf((_TOP - mu_x) * inv))


def kernel(mu, t, gamma, W1, b1, W2, b2):
    w1bf = W1[:_D].astype(jnp.bfloat16)
    w1r = W1[_D:]
    b1r = b1.reshape(1, _H)
    b2r = b2.reshape(1, 2 * _D)
    nd = _D // _DN

    h = pl.pallas_call(
        _mlp1_kernel,
        grid=(_B // _BM,),
        in_specs=[
            pl.BlockSpec((_BM, _D), lambda i: (i, 0)),
            pl.BlockSpec((_BM, 1), lambda i: (i, 0)),
            pl.BlockSpec((_D, _H), lambda i: (0, 0)),
            pl.BlockSpec((1, _H), lambda i: (0, 0)),
            pl.BlockSpec((1, _H), lambda i: (0, 0)),
        ],
        out_specs=pl.BlockSpec((_BM, _H), lambda i: (i, 0)),
        out_shape=jax.ShapeDtypeStruct((_B, _H), jnp.bfloat16),
        compiler_params=pltpu.CompilerParams(
            dimension_semantics=(_PAR,),
            vmem_limit_bytes=56 * 1024 * 1024,
        ),
    )(mu, t, w1bf, w1r, b1r)

    out = pl.pallas_call(
        _out_kernel,
        grid=(nd,),
        in_specs=[
            pl.BlockSpec((_B, _H), lambda j: (0, 0)),
            pl.BlockSpec((_H, _DN), lambda j: (0, j)),
            pl.BlockSpec((_H, _DN), lambda j: (0, j + nd)),
            pl.BlockSpec((1, _DN), lambda j: (0, j)),
            pl.BlockSpec((1, _DN), lambda j: (0, j + nd)),
            pl.BlockSpec((_B, _DN), lambda j: (0, j)),
            pl.BlockSpec((_B, 1), lambda j: (0, 0)),
        ],
        out_specs=pl.BlockSpec((_B, _DN), lambda j: (0, j)),
        out_shape=jax.ShapeDtypeStruct((_B, _D), jnp.float32),
        compiler_params=pltpu.CompilerParams(
            dimension_semantics=(_PAR,),
            vmem_limit_bytes=56 * 1024 * 1024,
        ),
    )(h, W2, W2, b2r, b2r, mu, gamma)

    return out


# speedup vs baseline: 4.8735x; 1.0126x over previous
import jax
import jax.numpy as jnp
from jax.experimental import pallas as pl
from jax.experimental.pallas import tpu as pltpu

_B, _D, _H, _K = 2048, 4096, 1024, 16
_SLOPE = 0.01
_TOP = 2.0 * (_K - 1.0) / _K - 1.0
_SQRT2 = 1.4142135623730951

_BM = 512
_DN = 512

_PAR = pltpu.GridDimensionSemantics.PARALLEL


def _mlp1_kernel(mu_ref, t_ref, g_ref, w1_ref, w1r_ref, b1_ref,
                 h_ref, rg_ref, s_ref, rs_ref):
    acc = jnp.dot(mu_ref[...].astype(jnp.bfloat16), w1_ref[...],
                  preferred_element_type=jnp.float32)
    acc = acc + t_ref[...] * w1r_ref[...] + b1_ref[...]
    h_ref[...] = jnp.where(acc >= 0.0, acc, _SLOPE * acc).astype(jnp.bfloat16)
    g = g_ref[...]
    s = jnp.sqrt((1.0 - g) / g)
    rg_ref[...] = 1.0 / g
    s_ref[...] = s
    rs_ref[...] = 1.0 / (s * _SQRT2)


def _out_kernel(h_ref, w2a_ref, w2b_ref, b2a_ref, b2b_ref, mu_ref,
                rg_ref, s_ref, rs_ref, o_ref):
    h = h_ref[...]
    mu_eps = jnp.dot(h, w2a_ref[...].astype(jnp.bfloat16),
                     preferred_element_type=jnp.float32) + b2a_ref[...]
    lse = jnp.dot(h, w2b_ref[...].astype(jnp.bfloat16),
                  preferred_element_type=jnp.float32) + b2b_ref[...]
    num = (_TOP - mu_ref[...] * rg_ref[...]) + s_ref[...] * mu_eps
    inv = rs_ref[...] * jnp.exp(-lse)
    o_ref[...] = 0.5 + 0.5 * jax.lax.erf(num * inv)


def kernel(mu, t, gamma, W1, b1, W2, b2):
    w1bf = W1[:_D].astype(jnp.bfloat16)
    w1r = W1[_D:]
    b1r = b1.reshape(1, _H)
    b2r = b2.reshape(1, 2 * _D)
    nd = _D // _DN

    rows = jax.ShapeDtypeStruct((_B, 1), jnp.float32)
    h, rg, s, rs = pl.pallas_call(
        _mlp1_kernel,
        grid=(_B // _BM,),
        in_specs=[
            pl.BlockSpec((_BM, _D), lambda i: (i, 0)),
            pl.BlockSpec((_BM, 1), lambda i: (i, 0)),
            pl.BlockSpec((_BM, 1), lambda i: (i, 0)),
            pl.BlockSpec((_D, _H), lambda i: (0, 0)),
            pl.BlockSpec((1, _H), lambda i: (0, 0)),
            pl.BlockSpec((1, _H), lambda i: (0, 0)),
        ],
        out_specs=[
            pl.BlockSpec((_BM, _H), lambda i: (i, 0)),
            pl.BlockSpec((_BM, 1), lambda i: (i, 0)),
            pl.BlockSpec((_BM, 1), lambda i: (i, 0)),
            pl.BlockSpec((_BM, 1), lambda i: (i, 0)),
        ],
        out_shape=[
            jax.ShapeDtypeStruct((_B, _H), jnp.bfloat16),
            rows, rows, rows,
        ],
        compiler_params=pltpu.CompilerParams(
            dimension_semantics=(_PAR,),
            vmem_limit_bytes=56 * 1024 * 1024,
        ),
    )(mu, t, gamma, w1bf, w1r, b1r)

    out = pl.pallas_call(
        _out_kernel,
        grid=(nd,),
        in_specs=[
            pl.BlockSpec((_B, _H), lambda j: (0, 0)),
            pl.BlockSpec((_H, _DN), lambda j: (0, j)),
            pl.BlockSpec((_H, _DN), lambda j: (0, j + nd)),
            pl.BlockSpec((1, _DN), lambda j: (0, j)),
            pl.BlockSpec((1, _DN), lambda j: (0, j + nd)),
            pl.BlockSpec((_B, _DN), lambda j: (0, j)),
            pl.BlockSpec((_B, 1), lambda j: (0, 0)),
            pl.BlockSpec((_B, 1), lambda j: (0, 0)),
            pl.BlockSpec((_B, 1), lambda j: (0, 0)),
        ],
        out_specs=pl.BlockSpec((_B, _DN), lambda j: (0, j)),
        out_shape=jax.ShapeDtypeStruct((_B, _D), jnp.float32),
        compiler_params=pltpu.CompilerParams(
            dimension_semantics=(_PAR,),
            vmem_limit_bytes=56 * 1024 * 1024,
        ),
    )(h, W2, W2, b2r, b2r, mu, rg, s, rs)

    return out


# speedup vs baseline: 5.2625x; 1.0798x over previous
import jax
import jax.numpy as jnp
from jax.experimental import pallas as pl
from jax.experimental.pallas import tpu as pltpu

_B, _D, _H, _K = 2048, 4096, 1024, 16
_SLOPE = 0.01
_TOP = 2.0 * (_K - 1.0) / _K - 1.0
_SQRT2 = 1.4142135623730951

_BM = 512
_DN = 512

_PAR = pltpu.GridDimensionSemantics.PARALLEL


def _mlp1_kernel(mu_ref, t_ref, g_ref, w1_ref, b1_ref,
                 h_ref, rg_ref, s_ref, rs_ref):
    acc = jnp.dot(mu_ref[...], w1_ref[0:_D, :],
                  preferred_element_type=jnp.float32)
    acc = acc + t_ref[...] * w1_ref[_D:_D + 1, :] + b1_ref[...]
    h_ref[...] = jnp.where(acc >= 0.0, acc, _SLOPE * acc)
    g = g_ref[...]
    s = jnp.sqrt((1.0 - g) / g)
    rg_ref[...] = 1.0 / g
    s_ref[...] = s
    rs_ref[...] = 1.0 / (s * _SQRT2)


def _out_kernel(h_ref, w2a_ref, w2b_ref, b2a_ref, b2b_ref, mu_ref,
                rg_ref, s_ref, rs_ref, o_ref):
    h = h_ref[...]
    mu_eps = jnp.dot(h, w2a_ref[...],
                     preferred_element_type=jnp.float32) + b2a_ref[...]
    lse = jnp.dot(h, w2b_ref[...],
                  preferred_element_type=jnp.float32) + b2b_ref[...]
    num = (_TOP - mu_ref[...] * rg_ref[...]) + s_ref[...] * mu_eps
    inv = rs_ref[...] * jnp.exp(-lse)
    o_ref[...] = 0.5 + 0.5 * jax.lax.erf(num * inv)


def kernel(mu, t, gamma, W1, b1, W2, b2):
    b1r = b1.reshape(1, _H)
    b2r = b2.reshape(1, 2 * _D)
    nd = _D // _DN

    rows = jax.ShapeDtypeStruct((_B, 1), jnp.float32)
    h, rg, s, rs = pl.pallas_call(
        _mlp1_kernel,
        grid=(_B // _BM,),
        in_specs=[
            pl.BlockSpec((_BM, _D), lambda i: (i, 0)),
            pl.BlockSpec((_BM, 1), lambda i: (i, 0)),
            pl.BlockSpec((_BM, 1), lambda i: (i, 0)),
            pl.BlockSpec((_D + 1, _H), lambda i: (0, 0)),
            pl.BlockSpec((1, _H), lambda i: (0, 0)),
        ],
        out_specs=[
            pl.BlockSpec((_BM, _H), lambda i: (i, 0)),
            pl.BlockSpec((_BM, 1), lambda i: (i, 0)),
            pl.BlockSpec((_BM, 1), lambda i: (i, 0)),
            pl.BlockSpec((_BM, 1), lambda i: (i, 0)),
        ],
        out_shape=[
            jax.ShapeDtypeStruct((_B, _H), jnp.float32),
            rows, rows, rows,
        ],
        compiler_params=pltpu.CompilerParams(
            dimension_semantics=(_PAR,),
            vmem_limit_bytes=56 * 1024 * 1024,
        ),
    )(mu, t, gamma, W1, b1r)

    out = pl.pallas_call(
        _out_kernel,
        grid=(nd,),
        in_specs=[
            pl.BlockSpec((_B, _H), lambda j: (0, 0)),
            pl.BlockSpec((_H, _DN), lambda j: (0, j)),
            pl.BlockSpec((_H, _DN), lambda j: (0, j + nd)),
            pl.BlockSpec((1, _DN), lambda j: (0, j)),
            pl.BlockSpec((1, _DN), lambda j: (0, j + nd)),
            pl.BlockSpec((_B, _DN), lambda j: (0, j)),
            pl.BlockSpec((_B, 1), lambda j: (0, 0)),
            pl.BlockSpec((_B, 1), lambda j: (0, 0)),
            pl.BlockSpec((_B, 1), lambda j: (0, 0)),
        ],
        out_specs=pl.BlockSpec((_B, _DN), lambda j: (0, j)),
        out_shape=jax.ShapeDtypeStruct((_B, _D), jnp.float32),
        compiler_params=pltpu.CompilerParams(
            dimension_semantics=(_PAR,),
            vmem_limit_bytes=56 * 1024 * 1024,
        ),
    )(h, W2, W2, b2r, b2r, mu, rg, s, rs)

    return out
